# baseline (device time: 71149 ns/iter reference)
import jax
import jax.numpy as jnp
from jax import lax
from jax.experimental import pallas as pl
from jax.experimental.pallas import tpu as pltpu

N_DEV = 32


def kernel(A, B):
    m, k_loc = A.shape
    _, n = B.shape
    chunk = m // N_DEV

    def body(a_ref, b_ref, out_ref, partial_ref, recv_ref, red_ref,
             send_a, recv_a, send_b, recv_b):
        me = lax.axis_index("i")

        a = a_ref[:, :].astype(jnp.bfloat16)
        b = b_ref[:, :].astype(jnp.bfloat16)
        part = jnp.dot(a, b, preferred_element_type=jnp.float32)
        partial_ref[:, :] = part.astype(jnp.bfloat16)

        p1 = []
        for o in range(1, N_DEV):
            tgt = (me + o) % N_DEV
            rdma = pltpu.make_async_remote_copy(
                src_ref=partial_ref.at[pl.ds(tgt * chunk, chunk), :],
                dst_ref=recv_ref.at[pl.ds(me * chunk, chunk), :],
                send_sem=send_a.at[o - 1],
                recv_sem=recv_a.at[o - 1],
                device_id=(tgt,),
                device_id_type=pl.DeviceIdType.MESH,
            )
            rdma.start()
            p1.append(rdma)

        recv_ref[pl.ds(me * chunk, chunk), :] = partial_ref[
            pl.ds(me * chunk, chunk), :
        ]

        for o in range(1, N_DEV):
            src = (me + o) % N_DEV
            w = pltpu.make_async_remote_copy(
                src_ref=partial_ref.at[pl.ds(0, chunk), :],
                dst_ref=recv_ref.at[pl.ds(src * chunk, chunk), :],
                send_sem=send_a.at[o - 1],
                recv_sem=recv_a.at[(N_DEV - o) - 1],
                device_id=(src,),
                device_id_type=pl.DeviceIdType.MESH,
            )
            w.wait_recv()

        stacked = recv_ref[:, :].reshape(N_DEV, chunk, n).astype(jnp.float32)
        red_ref[:, :] = jnp.sum(stacked, axis=0).astype(jnp.bfloat16)

        for r in p1:
            r.wait_send()

        p2 = []
        for o in range(1, N_DEV):
            tgt = (me + o) % N_DEV
            rdma = pltpu.make_async_remote_copy(
                src_ref=red_ref,
                dst_ref=out_ref.at[pl.ds(me * chunk, chunk), :],
                send_sem=send_b.at[o - 1],
                recv_sem=recv_b.at[o - 1],
                device_id=(tgt,),
                device_id_type=pl.DeviceIdType.MESH,
            )
            rdma.start()
            p2.append(rdma)

        out_ref[pl.ds(me * chunk, chunk), :] = red_ref[:, :]

        for o in range(1, N_DEV):
            src = (me + o) % N_DEV
            w = pltpu.make_async_remote_copy(
                src_ref=red_ref,
                dst_ref=out_ref.at[pl.ds(src * chunk, chunk), :],
                send_sem=send_b.at[o - 1],
                recv_sem=recv_b.at[(N_DEV - o) - 1],
                device_id=(src,),
                device_id_type=pl.DeviceIdType.MESH,
            )
            w.wait_recv()

        for r in p2:
            r.wait_send()

    return pl.pallas_call(
        body,
        out_shape=jax.ShapeDtypeStruct((m, n), jnp.bfloat16),
        in_specs=[
            pl.BlockSpec(memory_space=pltpu.VMEM),
            pl.BlockSpec(memory_space=pltpu.VMEM),
        ],
        out_specs=pl.BlockSpec(memory_space=pltpu.VMEM),
        scratch_shapes=[
            pltpu.VMEM((m, n), jnp.bfloat16),
            pltpu.VMEM((m, n), jnp.bfloat16),
            pltpu.VMEM((chunk, n), jnp.bfloat16),
            pltpu.SemaphoreType.DMA((N_DEV - 1,)),
            pltpu.SemaphoreType.DMA((N_DEV - 1,)),
            pltpu.SemaphoreType.DMA((N_DEV - 1,)),
            pltpu.SemaphoreType.DMA((N_DEV - 1,)),
        ],
    )(A, B)
